# baseline (device time: 500368 ns/iter reference)
import jax
import jax.numpy as jnp
from jax import lax
from jax.experimental import pallas as pl
from jax.experimental.pallas import tpu as pltpu


def _exchange(send):
    S, N = send.shape

    def body(send_ref, out_ref, send_sem, recv_sem):
        my_x = lax.axis_index("x")
        my_y = lax.axis_index("y")
        nbr = (my_x, 1 - my_y)

        barrier = pltpu.get_barrier_semaphore()
        pl.semaphore_signal(
            barrier, inc=1, device_id=nbr, device_id_type=pl.DeviceIdType.MESH
        )
        pl.semaphore_wait(barrier, 1)

        rdma = pltpu.make_async_remote_copy(
            src_ref=send_ref,
            dst_ref=out_ref,
            send_sem=send_sem,
            recv_sem=recv_sem,
            device_id=nbr,
            device_id_type=pl.DeviceIdType.MESH,
        )
        rdma.start()
        rdma.wait()

    return pl.pallas_call(
        body,
        out_shape=jax.ShapeDtypeStruct((S, N), send.dtype),
        in_specs=[pl.BlockSpec(memory_space=pltpu.VMEM)],
        out_specs=pl.BlockSpec(memory_space=pltpu.VMEM),
        scratch_shapes=[pltpu.SemaphoreType.DMA, pltpu.SemaphoreType.DMA],
        compiler_params=pltpu.CompilerParams(collective_id=0),
    )(send)


def kernel(O, Wo):
    B, S2, H, D = O.shape
    S = S2 // 2
    N = Wo.shape[1]

    A = O.reshape(S2, H * D).astype(jnp.bfloat16)
    W = Wo.astype(jnp.bfloat16)
    P = jnp.dot(A, W, preferred_element_type=jnp.float32)

    my_y = lax.axis_index("y")
    keep = lax.dynamic_slice_in_dim(P, my_y * S, S, axis=0)
    send = lax.dynamic_slice_in_dim(P, (1 - my_y) * S, S, axis=0)

    recv = _exchange(send.astype(jnp.bfloat16))
    return (keep + recv.astype(jnp.float32)).reshape(B, S, N)


# device time: 350373 ns/iter; 1.4281x vs baseline; 1.4281x over previous
import jax
import jax.numpy as jnp
from jax import lax
from jax.experimental import pallas as pl
from jax.experimental.pallas import tpu as pltpu

BN = 256
N_SEND_SLOTS = 4


def _fused(A, Wo, S):
    S2, K = A.shape
    N = Wo.shape[1]
    NB = N // BN

    def body(a_ref, w_hbm, out_ref, w_buf, send_buf, recv_buf,
             w_sems, send_sems, recv_sems):
        my_x = lax.axis_index("x")
        my_y = lax.axis_index("y")
        nbr = (my_x, 1 - my_y)
        keep_off = my_y * S
        send_off = (1 - my_y) * S

        def w_copy(j, slot):
            return pltpu.make_async_copy(
                w_hbm.at[:, pl.ds(j * BN, BN)], w_buf.at[slot], w_sems.at[slot]
            )

        def exchange(ssl, j):
            return pltpu.make_async_remote_copy(
                src_ref=send_buf.at[ssl],
                dst_ref=recv_buf.at[j],
                send_sem=send_sems.at[ssl],
                recv_sem=recv_sems.at[j],
                device_id=nbr,
                device_id_type=pl.DeviceIdType.MESH,
            )

        w_copy(0, 0).start()

        barrier = pltpu.get_barrier_semaphore()
        pl.semaphore_signal(
            barrier, inc=1, device_id=nbr, device_id_type=pl.DeviceIdType.MESH
        )
        pl.semaphore_wait(barrier, 1)

        def step(j, carry):
            slot = lax.rem(j, 2)
            ssl = lax.rem(j, N_SEND_SLOTS)

            @pl.when(j + 1 < NB)
            def _():
                w_copy(j + 1, lax.rem(j + 1, 2)).start()

            w_copy(j, slot).wait()
            w = w_buf[slot].astype(jnp.bfloat16)

            p_s = jax.lax.dot(
                a_ref[pl.ds(send_off, S), :], w,
                preferred_element_type=jnp.float32,
            )

            @pl.when(j >= N_SEND_SLOTS)
            def _():
                exchange(ssl, 0).wait_send()

            send_buf[ssl] = p_s.astype(jnp.bfloat16)
            exchange(ssl, j).start()

            p_k = jax.lax.dot(
                a_ref[pl.ds(keep_off, S), :], w,
                preferred_element_type=jnp.float32,
            )
            out_ref[0, :, pl.ds(j * BN, BN)] = p_k.astype(jnp.bfloat16)

            @pl.when(j >= 1)
            def _():
                jm = j - 1
                exchange(0, jm).wait_recv()
                out_ref[0, :, pl.ds(jm * BN, BN)] = (
                    out_ref[0, :, pl.ds(jm * BN, BN)] + recv_buf[jm]
                )

            return carry

        lax.fori_loop(0, NB, step, 0)

        exchange(0, NB - 1).wait_recv()
        out_ref[0, :, pl.ds((NB - 1) * BN, BN)] = (
            out_ref[0, :, pl.ds((NB - 1) * BN, BN)] + recv_buf[NB - 1]
        )
        for s in range(N_SEND_SLOTS):
            exchange(s, 0).wait_send()

    return pl.pallas_call(
        body,
        out_shape=jax.ShapeDtypeStruct((1, S, N), jnp.bfloat16),
        in_specs=[
            pl.BlockSpec(memory_space=pltpu.MemorySpace.VMEM),
            pl.BlockSpec(memory_space=pl.ANY),
        ],
        out_specs=pl.BlockSpec(memory_space=pltpu.MemorySpace.VMEM),
        scratch_shapes=[
            pltpu.VMEM((2, K, BN), jnp.float32),
            pltpu.VMEM((N_SEND_SLOTS, S, BN), jnp.bfloat16),
            pltpu.VMEM((NB, S, BN), jnp.bfloat16),
            pltpu.SemaphoreType.DMA((2,)),
            pltpu.SemaphoreType.DMA((N_SEND_SLOTS,)),
            pltpu.SemaphoreType.DMA((NB,)),
        ],
        compiler_params=pltpu.CompilerParams(
            collective_id=0,
            vmem_limit_bytes=64 * 1024 * 1024,
        ),
    )(A, Wo)


def kernel(O, Wo):
    B, S2, H, D = O.shape
    S = S2 // 2
    A = O.reshape(S2, H * D).astype(jnp.bfloat16)
    return _fused(A, Wo, S)
